# baseline (speedup 1.0000x reference)
import functools

import jax
import jax.numpy as jnp
from jax import lax
from jax.experimental import pallas as pl
from jax.experimental.pallas import tpu as pltpu
from jax.experimental.pallas import tpu_sc as plsc

_K = 33
_MIN32 = -2147483648
_ROWS = 128
_COLS = 32768
_CAP = 4096
_L = 16
_U = 8
_NW = 32


def _skey(v):
    b = lax.bitcast_convert_type(v, jnp.int32)
    return b ^ ((b >> 31) & jnp.int32(0x7FFFFFFF))


def _sc_body(sim_hbm, out_hbm, buf0, buf1, candi, si0, si1, so0, so1):
    nc = 2
    wid = lax.axis_index("s") * nc + lax.axis_index("c")
    iota = lax.iota(jnp.int32, _L)
    ninf = jnp.full((_L,), -jnp.inf, jnp.float32)
    n_chunks = _COLS // _L
    rows_per = _ROWS // _NW
    bufs = [buf0, buf1]
    in_sems = [si0, si1]
    out_sems = [so0, so1]

    def in_copy(j):
        return pltpu.make_async_copy(
            sim_hbm.at[wid * rows_per + j], bufs[j % 2], in_sems[j % 2])

    def out_copy(j):
        return pltpu.make_async_copy(
            bufs[j % 2], out_hbm.at[wid * rows_per + j], out_sems[j % 2])

    def compute(row_v):
        def pass_a(i, carry):
            ms = list(carry)
            for u in range(_U):
                s = u % 2
                x = row_v[pl.ds((i * _U + u) * _L, _L)]
                hi1 = jnp.maximum(ms[3 * s], x)
                lo1 = jnp.minimum(ms[3 * s], x)
                hi2 = jnp.maximum(ms[3 * s + 1], lo1)
                lo2 = jnp.minimum(ms[3 * s + 1], lo1)
                ms[3 * s] = hi1
                ms[3 * s + 1] = hi2
                ms[3 * s + 2] = jnp.maximum(ms[3 * s + 2], lo2)
            return tuple(ms)

        ms = lax.fori_loop(0, n_chunks // _U, pass_a, (ninf,) * 6)
        t0 = jnp.broadcast_to(jnp.min(jnp.minimum(ms[2], ms[5])), (_L,))

        def pass_b(i, cntv):
            msks = []
            orm = None
            for u in range(_U):
                c0 = (i * _U + u) * _L
                x = row_v[pl.ds(c0, _L)]
                msk = x >= t0
                row_v[pl.ds(c0, _L)] = jnp.where(msk, x, ninf)
                msks.append(msk)
                orm = msk if u == 0 else (orm | msk)

            def slow(c):
                for u in range(_U):
                    msk = msks[u]
                    c0 = (i * _U + u) * _L

                    def append(c2, msk=msk, c0=c0):
                        pos = c2 + plsc.cumsum(msk.astype(jnp.int32)) - 1
                        pos = jnp.minimum(pos, _CAP - 1)
                        plsc.store_scatter(candi, [pos], iota + c0,
                                           mask=msk)
                        return c2 + plsc.all_reduce_population_count(msk)

                    c = lax.cond(jnp.any(msk), append, lambda z: z, c)
                return c

            return lax.cond(jnp.any(orm), slow, lambda z: z, cntv)

        cntv = lax.fori_loop(0, n_chunks // _U, pass_b,
                             jnp.zeros((_L,), jnp.int32))
        cntv = jnp.minimum(cntv, _CAP)
        n_cand = jnp.max(cntv)
        nch = (n_cand + _L - 1) // _L

        def search_step(k, prefix_u):
            bit = jnp.int32(1) << (jnp.int32(31) - k)
            cand_u = prefix_u | bit
            cand_s = jnp.broadcast_to(cand_u ^ jnp.int32(_MIN32), (_L,))

            def count_chunk(ch, acc):
                b0 = ch * _L
                iv = candi[pl.ds(b0, _L)] & jnp.int32(_COLS - 1)
                sk = _skey(plsc.load_gather(row_v, [iv]))
                valid = (iota + b0) < cntv
                return acc + ((sk >= cand_s) & valid).astype(jnp.int32)

            acc = lax.fori_loop(0, nch, count_chunk,
                                jnp.zeros((_L,), jnp.int32))
            return jnp.where(jnp.sum(acc) >= _K, cand_u, prefix_u)

        prefix_u = lax.fori_loop(0, 32, search_step, jnp.int32(0))
        t_v = jnp.broadcast_to(prefix_u ^ jnp.int32(_MIN32), (_L,))

        def gt_chunk(ch, acc):
            b0 = ch * _L
            iv = candi[pl.ds(b0, _L)] & jnp.int32(_COLS - 1)
            sk = _skey(plsc.load_gather(row_v, [iv]))
            valid = (iota + b0) < cntv
            return acc + ((sk > t_v) & valid).astype(jnp.int32)

        n_gt = jnp.sum(lax.fori_loop(0, nch, gt_chunk,
                                     jnp.zeros((_L,), jnp.int32)))
        need = jnp.broadcast_to(_K - n_gt, (_L,))

        def fixup_chunk(ch, rank_base):
            b0 = ch * _L
            iv = candi[pl.ds(b0, _L)] & jnp.int32(_COLS - 1)
            sk = _skey(plsc.load_gather(row_v, [iv]))
            valid = (iota + b0) < cntv
            gtm = (sk > t_v) & valid
            eqm = (sk == t_v) & valid
            rank = jnp.broadcast_to(rank_base, (_L,)) + \
                plsc.cumsum(eqm.astype(jnp.int32))
            keep = gtm | (eqm & (rank <= need))
            rej = valid & jnp.logical_not(keep)
            plsc.store_scatter(row_v, [iv], ninf, mask=rej)
            return rank_base + jnp.sum(eqm.astype(jnp.int32))

        lax.fori_loop(0, nch, fixup_chunk, jnp.int32(0))

    in_copy(0).start()
    for j in range(rows_per):
        if j + 1 < rows_per:
            if j >= 1:
                out_copy(j - 1).wait()
            in_copy(j + 1).start()
        in_copy(j).wait()
        compute(bufs[j % 2])
        out_copy(j).start()
    out_copy(rows_per - 2).wait()
    out_copy(rows_per - 1).wait()


@jax.jit
def kernel(sim):
    mesh = plsc.VectorSubcoreMesh(core_axis_name="c", subcore_axis_name="s")
    f = functools.partial(
        pl.kernel,
        mesh=mesh,
        out_type=jax.ShapeDtypeStruct((_ROWS, _COLS), jnp.float32),
        scratch_types=[
            pltpu.VMEM((_COLS,), jnp.float32),
            pltpu.VMEM((_COLS,), jnp.float32),
            pltpu.VMEM((_CAP,), jnp.int32),
            pltpu.SemaphoreType.DMA,
            pltpu.SemaphoreType.DMA,
            pltpu.SemaphoreType.DMA,
            pltpu.SemaphoreType.DMA,
        ],
        compiler_params=pltpu.CompilerParams(needs_layout_passes=False),
    )(_sc_body)
    return f(sim)


# speedup vs baseline: 1.6130x; 1.6130x over previous
import functools

import jax
import jax.numpy as jnp
from jax import lax
from jax.experimental import pallas as pl
from jax.experimental.pallas import tpu as pltpu
from jax.experimental.pallas import tpu_sc as plsc

_K = 33
_MIN32 = -2147483648
_MAX32 = 2147483647
_ROWS = 128
_COLS = 32768
_STRIDE = 256
_CAP = _STRIDE * 16
_L = 16
_U = 8
_NW = 32

_DNUMS = lax.GatherDimensionNumbers(
    offset_dims=(), collapsed_slice_dims=(0,), start_index_map=(0,))


def _skey(v):
    b = lax.bitcast_convert_type(v, jnp.int32)
    return b ^ ((b >> 31) & jnp.int32(0x7FFFFFFF))


def _splat_last(v):
    idx = jnp.full((_L, 1), _L - 1, jnp.int32)
    return lax.gather(v, idx, _DNUMS, (1,),
                      mode=lax.GatherScatterMode.PROMISE_IN_BOUNDS)


def _splat_total(v_i32):
    return _splat_last(plsc.cumsum(v_i32))


def _splat_min_i32(v_i32):
    return -_splat_last(plsc.cummax(-v_i32))


def _sc_body(sim_hbm, out_hbm, buf0, buf1, candi, skbuf, ivbuf,
             si0, si1, so0, so1):
    nc = 2
    wid = lax.axis_index("s") * nc + lax.axis_index("c")
    iota = lax.iota(jnp.int32, _L)
    ninf = jnp.full((_L,), -jnp.inf, jnp.float32)
    n_chunks = _COLS // _L
    rows_per = _ROWS // _NW
    bufs = [buf0, buf1]
    in_sems = [si0, si1]
    out_sems = [so0, so1]
    lane_base = iota * _STRIDE
    capv = lane_base + (_STRIDE - 1)

    def in_copy(j):
        return pltpu.make_async_copy(
            sim_hbm.at[wid * rows_per + j], bufs[j % 2], in_sems[j % 2])

    def out_copy(j):
        return pltpu.make_async_copy(
            bufs[j % 2], out_hbm.at[wid * rows_per + j], out_sems[j % 2])

    def compute(row_v):
        def pass_a(i, carry):
            ms = list(carry)
            for u in range(_U):
                s = u % 4
                x = row_v[pl.ds((i * _U + u) * _L, _L)]
                hi = jnp.maximum(ms[2 * s], x)
                lo = jnp.minimum(ms[2 * s], x)
                ms[2 * s] = hi
                ms[2 * s + 1] = jnp.maximum(ms[2 * s + 1], lo)
            return tuple(ms)

        ms = lax.fori_loop(0, n_chunks // _U, pass_a, (ninf,) * 8)
        m2min = jnp.minimum(jnp.minimum(ms[1], ms[3]),
                            jnp.minimum(ms[5], ms[7]))
        t0k = _splat_min_i32(_skey(m2min))
        t0 = lax.bitcast_convert_type(
            t0k ^ ((t0k >> 31) & jnp.int32(0x7FFFFFFF)), jnp.float32)

        def pass_b(i, posv):
            for u in range(_U):
                c0 = (i * _U + u) * _L
                x = row_v[pl.ds(c0, _L)]
                msk = x >= t0
                row_v[pl.ds(c0, _L)] = jnp.where(msk, x, ninf)
                plsc.store_scatter(candi, [jnp.minimum(posv, capv)],
                                   iota + c0, mask=msk)
                posv = posv + msk.astype(jnp.int32)
            return posv

        posv = lax.fori_loop(0, n_chunks // _U, pass_b, lane_base)
        cnt_lane = jnp.minimum(posv - lane_base, _STRIDE - 1)
        nch_seg = jnp.max(cnt_lane)

        def compact(ch, base):
            iv = plsc.load_gather(candi, [lane_base + ch]) & \
                jnp.int32(_COLS - 1)
            validm = cnt_lane > ch
            sk = _skey(plsc.load_gather(row_v, [iv], mask=validm))
            pc = plsc.cumsum(validm.astype(jnp.int32))
            pos = jnp.minimum(base + pc - 1, _CAP - 1)
            plsc.store_scatter(skbuf, [pos], sk, mask=validm)
            plsc.store_scatter(ivbuf, [pos], iv, mask=validm)
            return base + _splat_last(pc)

        n_splat = lax.fori_loop(0, nch_seg, compact,
                                jnp.zeros((_L,), jnp.int32))
        nch = (jnp.max(n_splat) + _L - 1) // _L

        def valid_at(ch):
            return (iota + ch * _L) < n_splat

        def search_step(k, prefix_v):
            bit = jnp.int32(1) << (jnp.int32(31) - k)
            cand_v = prefix_v | bit
            cand_s = cand_v ^ jnp.int32(_MIN32)

            def count_chunk(ch, acc):
                sk = skbuf[pl.ds(ch * _L, _L)]
                return acc + ((sk >= cand_s) & valid_at(ch)).astype(
                    jnp.int32)

            acc = lax.fori_loop(0, nch, count_chunk,
                                jnp.zeros((_L,), jnp.int32))
            tot = _splat_total(acc)
            return jnp.where(tot >= _K, cand_v, prefix_v)

        prefix_v = lax.fori_loop(0, 32, search_step,
                                 jnp.zeros((_L,), jnp.int32))
        t_v = prefix_v ^ jnp.int32(_MIN32)

        def gteq_chunk(ch, carry):
            gt_acc, eq_acc = carry
            sk = skbuf[pl.ds(ch * _L, _L)]
            va = valid_at(ch)
            return (gt_acc + ((sk > t_v) & va).astype(jnp.int32),
                    eq_acc + ((sk == t_v) & va).astype(jnp.int32))

        zero = jnp.zeros((_L,), jnp.int32)
        n_gt, n_eq = lax.fori_loop(0, nch, gteq_chunk, (zero, zero))
        n_gt = _splat_total(n_gt)
        n_eq = _splat_total(n_eq)
        need = _K - n_gt
        s_rare = jnp.max(n_eq - need)

        @pl.when(s_rare == 0)
        def _():
            def promote_all(ch, _c):
                sk = skbuf[pl.ds(ch * _L, _L)]
                skbuf[pl.ds(ch * _L, _L)] = jnp.where(
                    (sk == t_v) & valid_at(ch), t_v + 1, sk)
                return 0

            lax.fori_loop(0, nch, promote_all, 0)

        @pl.when(s_rare != 0)
        def _():
            def promote_one(_r, _c):
                def min_chunk(ch, macc):
                    sk = skbuf[pl.ds(ch * _L, _L)]
                    ivv = ivbuf[pl.ds(ch * _L, _L)]
                    m = (sk == t_v) & valid_at(ch)
                    return jnp.minimum(
                        macc, jnp.where(m, ivv, jnp.int32(_MAX32)))

                macc = lax.fori_loop(0, nch, min_chunk,
                                     jnp.full((_L,), _MAX32, jnp.int32))
                mins = _splat_min_i32(macc)

                def mark_chunk(ch, _c2):
                    sk = skbuf[pl.ds(ch * _L, _L)]
                    ivv = ivbuf[pl.ds(ch * _L, _L)]
                    match = (ivv == mins) & (sk == t_v) & valid_at(ch)
                    skbuf[pl.ds(ch * _L, _L)] = jnp.where(
                        match, t_v + 1, sk)
                    return 0

                lax.fori_loop(0, nch, mark_chunk, 0)
                return 0

            lax.fori_loop(0, jnp.max(need), promote_one, 0)

        def fixup_chunk(ch, _c):
            sk = skbuf[pl.ds(ch * _L, _L)]
            iv = ivbuf[pl.ds(ch * _L, _L)] & jnp.int32(_COLS - 1)
            rej = valid_at(ch) & (sk <= t_v)
            plsc.store_scatter(row_v, [iv], ninf, mask=rej)
            return 0

        lax.fori_loop(0, nch, fixup_chunk, 0)

    in_copy(0).start()
    for j in range(rows_per):
        if j + 1 < rows_per:
            if j >= 1:
                out_copy(j - 1).wait()
            in_copy(j + 1).start()
        in_copy(j).wait()
        compute(bufs[j % 2])
        out_copy(j).start()
    out_copy(rows_per - 2).wait()
    out_copy(rows_per - 1).wait()


@jax.jit
def kernel(sim):
    mesh = plsc.VectorSubcoreMesh(core_axis_name="c", subcore_axis_name="s")
    f = functools.partial(
        pl.kernel,
        mesh=mesh,
        out_type=jax.ShapeDtypeStruct((_ROWS, _COLS), jnp.float32),
        scratch_types=[
            pltpu.VMEM((_COLS,), jnp.float32),
            pltpu.VMEM((_COLS,), jnp.float32),
            pltpu.VMEM((_CAP,), jnp.int32),
            pltpu.VMEM((_CAP,), jnp.int32),
            pltpu.VMEM((_CAP,), jnp.int32),
            pltpu.SemaphoreType.DMA,
            pltpu.SemaphoreType.DMA,
            pltpu.SemaphoreType.DMA,
            pltpu.SemaphoreType.DMA,
        ],
        compiler_params=pltpu.CompilerParams(needs_layout_passes=False),
    )(_sc_body)
    return f(sim)


# speedup vs baseline: 2.7594x; 1.7107x over previous
import functools

import jax
import jax.numpy as jnp
from jax import lax
from jax.experimental import pallas as pl
from jax.experimental.pallas import tpu as pltpu
from jax.experimental.pallas import tpu_sc as plsc

_K = 33
_MIN32 = -2147483648
_MAX32 = 2147483647
_ROWS = 128
_COLS = 32768
_STRIDE = 256
_CAP = _STRIDE * 16
_CAP2 = 512
_L = 16
_U = 8
_NW = 32

_DNUMS = lax.GatherDimensionNumbers(
    offset_dims=(), collapsed_slice_dims=(0,), start_index_map=(0,))


def _skey(v):
    b = lax.bitcast_convert_type(v, jnp.int32)
    return b ^ ((b >> 31) & jnp.int32(0x7FFFFFFF))


def _splat_last(v):
    idx = jnp.full((_L, 1), _L - 1, jnp.int32)
    return lax.gather(v, idx, _DNUMS, (1,),
                      mode=lax.GatherScatterMode.PROMISE_IN_BOUNDS)


def _splat_total(v_i32):
    return _splat_last(plsc.cumsum(v_i32))


def _splat_min_i32(v_i32):
    return -_splat_last(plsc.cummax(-v_i32))


def _sc_body(sim_hbm, out_hbm, buf0, buf1, candi, skbuf, ivbuf, skb2,
             si0, si1, so0, so1):
    nc = 2
    wid = lax.axis_index("s") * nc + lax.axis_index("c")
    iota = lax.iota(jnp.int32, _L)
    ninf = jnp.full((_L,), -jnp.inf, jnp.float32)
    n_chunks = _COLS // _L
    rows_per = _ROWS // _NW
    bufs = [buf0, buf1]
    in_sems = [si0, si1]
    out_sems = [so0, so1]
    lane_base = iota * _STRIDE
    capv = lane_base + (_STRIDE - 1)

    def in_copy(j):
        return pltpu.make_async_copy(
            sim_hbm.at[wid * rows_per + j], bufs[j % 2], in_sems[j % 2])

    def out_copy(j):
        return pltpu.make_async_copy(
            bufs[j % 2], out_hbm.at[wid * rows_per + j], out_sems[j % 2])

    def compute(row_v):
        def pass_a(i, carry):
            ms = list(carry)
            for u in range(_U):
                s = u % 4
                x = row_v[pl.ds((i * _U + u) * _L, _L)]
                hi = jnp.maximum(ms[2 * s], x)
                lo = jnp.minimum(ms[2 * s], x)
                ms[2 * s] = hi
                ms[2 * s + 1] = jnp.maximum(ms[2 * s + 1], lo)
            return tuple(ms)

        ms = lax.fori_loop(0, n_chunks // _U, pass_a, (ninf,) * 8)
        m2min = jnp.minimum(jnp.minimum(ms[1], ms[3]),
                            jnp.minimum(ms[5], ms[7]))
        t0k = _splat_min_i32(_skey(m2min))
        t0 = lax.bitcast_convert_type(
            t0k ^ ((t0k >> 31) & jnp.int32(0x7FFFFFFF)), jnp.float32)

        def pass_b(i, posv):
            msks = []
            incs = []
            for u in range(_U):
                c0 = (i * _U + u) * _L
                x = row_v[pl.ds(c0, _L)]
                msk = x >= t0
                row_v[pl.ds(c0, _L)] = jnp.where(msk, x, ninf)
                msks.append(msk)
                incs.append(msk.astype(jnp.int32))
            run = jnp.zeros((_L,), jnp.int32)
            for u in range(_U):
                c0 = (i * _U + u) * _L
                plsc.store_scatter(
                    candi, [jnp.minimum(posv + run, capv)],
                    iota + c0, mask=msks[u])
                run = run + incs[u]
            return posv + run

        posv = lax.fori_loop(0, n_chunks // _U, pass_b, lane_base)
        cnt_lane = jnp.minimum(posv - lane_base, _STRIDE - 1)
        nch_seg = jnp.max(cnt_lane)

        def compact(ch, base):
            iv = plsc.load_gather(candi, [lane_base + ch]) & \
                jnp.int32(_COLS - 1)
            validm = cnt_lane > ch
            sk = _skey(plsc.load_gather(row_v, [iv], mask=validm))
            pc = plsc.cumsum(validm.astype(jnp.int32))
            pos = jnp.minimum(base + pc - 1, _CAP - 1)
            plsc.store_scatter(skbuf, [pos], sk, mask=validm)
            plsc.store_scatter(ivbuf, [pos], iv, mask=validm)
            return base + _splat_last(pc)

        n_splat = lax.fori_loop(0, nch_seg, compact,
                                jnp.zeros((_L,), jnp.int32))
        nch = (jnp.max(n_splat) + _L - 1) // _L

        def valid_at(ch):
            return (iota + ch * _L) < n_splat

        def narrow_chunk(ch, carry):
            m1, m2, m3 = carry
            sk = jnp.where(valid_at(ch), skbuf[pl.ds(ch * _L, _L)],
                           jnp.int32(_MIN32))
            hi1 = jnp.maximum(m1, sk)
            lo1 = jnp.minimum(m1, sk)
            hi2 = jnp.maximum(m2, lo1)
            lo2 = jnp.minimum(m2, lo1)
            return hi1, hi2, jnp.maximum(m3, lo2)

        minv = jnp.full((_L,), _MIN32, jnp.int32)
        _m1, _m2, m3 = lax.fori_loop(0, nch, narrow_chunk,
                                     (minv, minv, minv))
        t1k = _splat_min_i32(m3)

        def recompact(ch, base):
            sk = skbuf[pl.ds(ch * _L, _L)]
            v2 = (sk >= t1k) & valid_at(ch)
            pc = plsc.cumsum(v2.astype(jnp.int32))
            pos = jnp.minimum(base + pc - 1, _CAP2 - 1)
            plsc.store_scatter(skb2, [pos], sk, mask=v2)
            return base + _splat_last(pc)

        n2_splat = lax.fori_loop(0, nch, recompact,
                                 jnp.zeros((_L,), jnp.int32))
        nch2 = (jnp.max(n2_splat) + _L - 1) // _L

        def valid2_at(ch):
            return (iota + ch * _L) < n2_splat

        def search_step(k, prefix_v):
            bit = jnp.int32(1) << (jnp.int32(31) - k)
            cand_v = prefix_v | bit
            cand_s = cand_v ^ jnp.int32(_MIN32)

            def count_chunk(ch, acc):
                sk = skb2[pl.ds(ch * _L, _L)]
                return acc + ((sk >= cand_s) & valid2_at(ch)).astype(
                    jnp.int32)

            acc = lax.fori_loop(0, nch2, count_chunk,
                                jnp.zeros((_L,), jnp.int32))
            tot = _splat_total(acc)
            return jnp.where(tot >= _K, cand_v, prefix_v)

        prefix_v = lax.fori_loop(0, 32, search_step,
                                 jnp.zeros((_L,), jnp.int32))
        t_v = prefix_v ^ jnp.int32(_MIN32)

        def gteq_chunk(ch, carry):
            gt_acc, eq_acc = carry
            sk = skb2[pl.ds(ch * _L, _L)]
            va = valid2_at(ch)
            return (gt_acc + ((sk > t_v) & va).astype(jnp.int32),
                    eq_acc + ((sk == t_v) & va).astype(jnp.int32))

        zero = jnp.zeros((_L,), jnp.int32)
        n_gt, n_eq = lax.fori_loop(0, nch2, gteq_chunk, (zero, zero))
        n_gt = _splat_total(n_gt)
        n_eq = _splat_total(n_eq)
        need = _K - n_gt
        s_rare = jnp.max(n_eq - need)

        @pl.when(s_rare == 0)
        def _():
            def promote_all(ch, _c):
                sk = skbuf[pl.ds(ch * _L, _L)]
                skbuf[pl.ds(ch * _L, _L)] = jnp.where(
                    (sk == t_v) & valid_at(ch), t_v + 1, sk)
                return 0

            lax.fori_loop(0, nch, promote_all, 0)

        @pl.when(s_rare != 0)
        def _():
            def promote_one(_r, _c):
                def min_chunk(ch, macc):
                    sk = skbuf[pl.ds(ch * _L, _L)]
                    ivv = ivbuf[pl.ds(ch * _L, _L)]
                    m = (sk == t_v) & valid_at(ch)
                    return jnp.minimum(
                        macc, jnp.where(m, ivv, jnp.int32(_MAX32)))

                macc = lax.fori_loop(0, nch, min_chunk,
                                     jnp.full((_L,), _MAX32, jnp.int32))
                mins = _splat_min_i32(macc)

                def mark_chunk(ch, _c2):
                    sk = skbuf[pl.ds(ch * _L, _L)]
                    ivv = ivbuf[pl.ds(ch * _L, _L)]
                    match = (ivv == mins) & (sk == t_v) & valid_at(ch)
                    skbuf[pl.ds(ch * _L, _L)] = jnp.where(
                        match, t_v + 1, sk)
                    return 0

                lax.fori_loop(0, nch, mark_chunk, 0)
                return 0

            lax.fori_loop(0, jnp.max(need), promote_one, 0)

        def fixup_chunk(ch, _c):
            sk = skbuf[pl.ds(ch * _L, _L)]
            iv = ivbuf[pl.ds(ch * _L, _L)] & jnp.int32(_COLS - 1)
            rej = valid_at(ch) & (sk <= t_v)
            plsc.store_scatter(row_v, [iv], ninf, mask=rej)
            return 0

        lax.fori_loop(0, nch, fixup_chunk, 0)

    in_copy(0).start()
    for j in range(rows_per):
        if j + 1 < rows_per:
            if j >= 1:
                out_copy(j - 1).wait()
            in_copy(j + 1).start()
        in_copy(j).wait()
        compute(bufs[j % 2])
        out_copy(j).start()
    out_copy(rows_per - 2).wait()
    out_copy(rows_per - 1).wait()


@jax.jit
def kernel(sim):
    mesh = plsc.VectorSubcoreMesh(core_axis_name="c", subcore_axis_name="s")
    f = functools.partial(
        pl.kernel,
        mesh=mesh,
        out_type=jax.ShapeDtypeStruct((_ROWS, _COLS), jnp.float32),
        scratch_types=[
            pltpu.VMEM((_COLS,), jnp.float32),
            pltpu.VMEM((_COLS,), jnp.float32),
            pltpu.VMEM((_CAP,), jnp.int32),
            pltpu.VMEM((_CAP,), jnp.int32),
            pltpu.VMEM((_CAP,), jnp.int32),
            pltpu.VMEM((_CAP2,), jnp.int32),
            pltpu.SemaphoreType.DMA,
            pltpu.SemaphoreType.DMA,
            pltpu.SemaphoreType.DMA,
            pltpu.SemaphoreType.DMA,
        ],
        compiler_params=pltpu.CompilerParams(needs_layout_passes=False),
    )(_sc_body)
    return f(sim)


# speedup vs baseline: 2.8303x; 1.0257x over previous
import functools

import jax
import jax.numpy as jnp
from jax import lax
from jax.experimental import pallas as pl
from jax.experimental.pallas import tpu as pltpu
from jax.experimental.pallas import tpu_sc as plsc

_K = 33
_MIN32 = -2147483648
_MAX32 = 2147483647
_ROWS = 128
_COLS = 32768
_STRIDE = 256
_CAP = _STRIDE * 16
_CAP2 = 512
_L = 16
_U = 8
_NW = 32

_DNUMS = lax.GatherDimensionNumbers(
    offset_dims=(), collapsed_slice_dims=(0,), start_index_map=(0,))


def _skey(v):
    b = lax.bitcast_convert_type(v, jnp.int32)
    return b ^ ((b >> 31) & jnp.int32(0x7FFFFFFF))


def _splat_last(v):
    idx = jnp.full((_L, 1), _L - 1, jnp.int32)
    return lax.gather(v, idx, _DNUMS, (1,),
                      mode=lax.GatherScatterMode.PROMISE_IN_BOUNDS)


def _splat_total(v_i32):
    return _splat_last(plsc.cumsum(v_i32))


def _splat_min_i32(v_i32):
    return -_splat_last(plsc.cummax(-v_i32))


def _sc_body(sim_hbm, out_hbm, buf0, buf1, candi, skbuf, ivbuf, skb2,
             si0, si1, so0, so1):
    nc = 2
    wid = lax.axis_index("s") * nc + lax.axis_index("c")
    iota = lax.iota(jnp.int32, _L)
    ninf = jnp.full((_L,), -jnp.inf, jnp.float32)
    n_chunks = _COLS // _L
    rows_per = _ROWS // _NW
    bufs = [buf0, buf1]
    in_sems = [si0, si1]
    out_sems = [so0, so1]
    lane_base = iota * _STRIDE
    capv = lane_base + (_STRIDE - 1)

    def in_copy(j):
        return pltpu.make_async_copy(
            sim_hbm.at[wid * rows_per + j], bufs[j % 2], in_sems[j % 2])

    def out_copy(j):
        return pltpu.make_async_copy(
            bufs[j % 2], out_hbm.at[wid * rows_per + j], out_sems[j % 2])

    def compute(row_v):
        def pass_a(i, carry):
            ms = list(carry)
            for u in range(_U):
                s = u % 4
                x = row_v[pl.ds((i * _U + u) * _L, _L)]
                hi = jnp.maximum(ms[2 * s], x)
                lo = jnp.minimum(ms[2 * s], x)
                ms[2 * s] = hi
                ms[2 * s + 1] = jnp.maximum(ms[2 * s + 1], lo)
            return tuple(ms)

        ms = lax.fori_loop(0, n_chunks // _U, pass_a, (ninf,) * 8)
        m2min = jnp.minimum(jnp.minimum(ms[1], ms[3]),
                            jnp.minimum(ms[5], ms[7]))
        t0k = _splat_min_i32(_skey(m2min))
        t0 = lax.bitcast_convert_type(
            t0k ^ ((t0k >> 31) & jnp.int32(0x7FFFFFFF)), jnp.float32)

        def pass_b(i, posv):
            base = jnp.minimum(posv, capv - _U)
            msks = []
            incs = []
            for u in range(_U):
                c0 = (i * _U + u) * _L
                x = row_v[pl.ds(c0, _L)]
                msk = x >= t0
                row_v[pl.ds(c0, _L)] = jnp.where(msk, x, ninf)
                msks.append(msk)
                incs.append(msk.astype(jnp.int32))
            run = jnp.zeros((_L,), jnp.int32)
            for u in range(_U):
                c0 = (i * _U + u) * _L
                plsc.store_scatter(candi, [base + run], iota + c0,
                                   mask=msks[u])
                run = run + incs[u]
            return base + run

        posv = lax.fori_loop(0, n_chunks // _U, pass_b, lane_base)
        cnt_lane = jnp.minimum(posv - lane_base, _STRIDE - 1)
        nch_seg = jnp.max(cnt_lane)

        def compact(ch, base):
            iv = plsc.load_gather(candi, [lane_base + ch]) & \
                jnp.int32(_COLS - 1)
            validm = cnt_lane > ch
            sk = _skey(plsc.load_gather(row_v, [iv], mask=validm))
            pc = plsc.cumsum(validm.astype(jnp.int32))
            pos = jnp.minimum(base + pc - 1, _CAP - 1)
            plsc.store_scatter(skbuf, [pos], sk, mask=validm)
            plsc.store_scatter(ivbuf, [pos], iv, mask=validm)
            return base + _splat_last(pc)

        n_splat = lax.fori_loop(0, nch_seg, compact,
                                jnp.zeros((_L,), jnp.int32))
        nch = (jnp.max(n_splat) + _L - 1) // _L

        def valid_at(ch):
            return (iota + ch * _L) < n_splat

        def narrow_chunk(ch, carry):
            m1, m2, m3 = carry
            sk = jnp.where(valid_at(ch), skbuf[pl.ds(ch * _L, _L)],
                           jnp.int32(_MIN32))
            hi1 = jnp.maximum(m1, sk)
            lo1 = jnp.minimum(m1, sk)
            hi2 = jnp.maximum(m2, lo1)
            lo2 = jnp.minimum(m2, lo1)
            return hi1, hi2, jnp.maximum(m3, lo2)

        minv = jnp.full((_L,), _MIN32, jnp.int32)
        _m1, _m2, m3 = lax.fori_loop(0, nch, narrow_chunk,
                                     (minv, minv, minv))
        t1k = _splat_min_i32(m3)

        def recompact(ch, base):
            sk = skbuf[pl.ds(ch * _L, _L)]
            v2 = (sk >= t1k) & valid_at(ch)
            pc = plsc.cumsum(v2.astype(jnp.int32))
            pos = jnp.minimum(base + pc - 1, _CAP2 - 1)
            plsc.store_scatter(skb2, [pos], sk, mask=v2)
            return base + _splat_last(pc)

        n2_splat = lax.fori_loop(0, nch, recompact,
                                 jnp.zeros((_L,), jnp.int32))
        nch2 = (jnp.max(n2_splat) + _L - 1) // _L

        def valid2_at(ch):
            return (iota + ch * _L) < n2_splat

        def search_step(k, prefix_v):
            bit = jnp.int32(1) << (jnp.int32(31) - k)
            cand_v = prefix_v | bit
            cand_s = cand_v ^ jnp.int32(_MIN32)

            def count_chunk(ch, acc):
                sk = skb2[pl.ds(ch * _L, _L)]
                return acc + ((sk >= cand_s) & valid2_at(ch)).astype(
                    jnp.int32)

            acc = lax.fori_loop(0, nch2, count_chunk,
                                jnp.zeros((_L,), jnp.int32))
            tot = _splat_total(acc)
            return jnp.where(tot >= _K, cand_v, prefix_v)

        prefix_v = lax.fori_loop(0, 32, search_step,
                                 jnp.zeros((_L,), jnp.int32))
        t_v = prefix_v ^ jnp.int32(_MIN32)

        def gteq_chunk(ch, carry):
            gt_acc, eq_acc = carry
            sk = skb2[pl.ds(ch * _L, _L)]
            va = valid2_at(ch)
            return (gt_acc + ((sk > t_v) & va).astype(jnp.int32),
                    eq_acc + ((sk == t_v) & va).astype(jnp.int32))

        zero = jnp.zeros((_L,), jnp.int32)
        n_gt, n_eq = lax.fori_loop(0, nch2, gteq_chunk, (zero, zero))
        n_gt = _splat_total(n_gt)
        n_eq = _splat_total(n_eq)
        need = _K - n_gt
        s_rare = jnp.max(n_eq - need)

        @pl.when(s_rare == 0)
        def _():
            def promote_all(ch, _c):
                sk = skbuf[pl.ds(ch * _L, _L)]
                skbuf[pl.ds(ch * _L, _L)] = jnp.where(
                    (sk == t_v) & valid_at(ch), t_v + 1, sk)
                return 0

            lax.fori_loop(0, nch, promote_all, 0)

        @pl.when(s_rare != 0)
        def _():
            def promote_one(_r, _c):
                def min_chunk(ch, macc):
                    sk = skbuf[pl.ds(ch * _L, _L)]
                    ivv = ivbuf[pl.ds(ch * _L, _L)]
                    m = (sk == t_v) & valid_at(ch)
                    return jnp.minimum(
                        macc, jnp.where(m, ivv, jnp.int32(_MAX32)))

                macc = lax.fori_loop(0, nch, min_chunk,
                                     jnp.full((_L,), _MAX32, jnp.int32))
                mins = _splat_min_i32(macc)

                def mark_chunk(ch, _c2):
                    sk = skbuf[pl.ds(ch * _L, _L)]
                    ivv = ivbuf[pl.ds(ch * _L, _L)]
                    match = (ivv == mins) & (sk == t_v) & valid_at(ch)
                    skbuf[pl.ds(ch * _L, _L)] = jnp.where(
                        match, t_v + 1, sk)
                    return 0

                lax.fori_loop(0, nch, mark_chunk, 0)
                return 0

            lax.fori_loop(0, jnp.max(need), promote_one, 0)

        def fixup_chunk(ch, _c):
            sk = skbuf[pl.ds(ch * _L, _L)]
            iv = ivbuf[pl.ds(ch * _L, _L)] & jnp.int32(_COLS - 1)
            rej = valid_at(ch) & (sk <= t_v)
            plsc.store_scatter(row_v, [iv], ninf, mask=rej)
            return 0

        lax.fori_loop(0, nch, fixup_chunk, 0)

    in_copy(0).start()
    for j in range(rows_per):
        if j + 1 < rows_per:
            if j >= 1:
                out_copy(j - 1).wait()
            in_copy(j + 1).start()
        in_copy(j).wait()
        compute(bufs[j % 2])
        out_copy(j).start()
    out_copy(rows_per - 2).wait()
    out_copy(rows_per - 1).wait()


@jax.jit
def kernel(sim):
    mesh = plsc.VectorSubcoreMesh(core_axis_name="c", subcore_axis_name="s")
    f = functools.partial(
        pl.kernel,
        mesh=mesh,
        out_type=jax.ShapeDtypeStruct((_ROWS, _COLS), jnp.float32),
        scratch_types=[
            pltpu.VMEM((_COLS,), jnp.float32),
            pltpu.VMEM((_COLS,), jnp.float32),
            pltpu.VMEM((_CAP,), jnp.int32),
            pltpu.VMEM((_CAP,), jnp.int32),
            pltpu.VMEM((_CAP,), jnp.int32),
            pltpu.VMEM((_CAP2,), jnp.int32),
            pltpu.SemaphoreType.DMA,
            pltpu.SemaphoreType.DMA,
            pltpu.SemaphoreType.DMA,
            pltpu.SemaphoreType.DMA,
        ],
        compiler_params=pltpu.CompilerParams(needs_layout_passes=False),
    )(_sc_body)
    return f(sim)


# speedup vs baseline: 2.9179x; 1.0309x over previous
import functools

import jax
import jax.numpy as jnp
from jax import lax
from jax.experimental import pallas as pl
from jax.experimental.pallas import tpu as pltpu
from jax.experimental.pallas import tpu_sc as plsc

_K = 33
_MIN32 = -2147483648
_MAX32 = 2147483647
_ROWS = 128
_COLS = 32768
_STRIDE = 256
_CAP = _STRIDE * 16
_CAP2 = 512
_L = 16
_U = 8
_UB = 16
_NW = 32

_DNUMS = lax.GatherDimensionNumbers(
    offset_dims=(), collapsed_slice_dims=(0,), start_index_map=(0,))


def _skey(v):
    b = lax.bitcast_convert_type(v, jnp.int32)
    return b ^ ((b >> 31) & jnp.int32(0x7FFFFFFF))


def _splat_last(v):
    idx = jnp.full((_L, 1), _L - 1, jnp.int32)
    return lax.gather(v, idx, _DNUMS, (1,),
                      mode=lax.GatherScatterMode.PROMISE_IN_BOUNDS)


def _splat_total(v_i32):
    return _splat_last(plsc.cumsum(v_i32))


def _splat_min_i32(v_i32):
    return -_splat_last(plsc.cummax(-v_i32))


def _sc_body(sim_hbm, out_hbm, buf0, buf1, candi, skbuf, ivbuf, skb2,
             si0, si1, so0, so1):
    nc = 2
    wid = lax.axis_index("s") * nc + lax.axis_index("c")
    iota = lax.iota(jnp.int32, _L)
    ninf = jnp.full((_L,), -jnp.inf, jnp.float32)
    n_chunks = _COLS // _L
    rows_per = _ROWS // _NW
    bufs = [buf0, buf1]
    in_sems = [si0, si1]
    out_sems = [so0, so1]
    lane_base = iota * _STRIDE
    capv = lane_base + (_STRIDE - 1)

    def in_copy(j):
        return pltpu.make_async_copy(
            sim_hbm.at[wid * rows_per + j], bufs[j % 2], in_sems[j % 2])

    def out_copy(j):
        return pltpu.make_async_copy(
            bufs[j % 2], out_hbm.at[wid * rows_per + j], out_sems[j % 2])

    def compute(row_v):
        def pass_a(i, carry):
            ms = list(carry)
            for u in range(_U):
                s = u % 4
                x = row_v[pl.ds((i * _U + u) * _L, _L)]
                hi = jnp.maximum(ms[2 * s], x)
                lo = jnp.minimum(ms[2 * s], x)
                ms[2 * s] = hi
                ms[2 * s + 1] = jnp.maximum(ms[2 * s + 1], lo)
            return tuple(ms)

        ms = lax.fori_loop(0, n_chunks // _U, pass_a, (ninf,) * 8)
        m2min = jnp.minimum(jnp.minimum(ms[1], ms[3]),
                            jnp.minimum(ms[5], ms[7]))
        t0k = _splat_min_i32(_skey(m2min))
        t0 = lax.bitcast_convert_type(
            t0k ^ ((t0k >> 31) & jnp.int32(0x7FFFFFFF)), jnp.float32)

        def pass_b(i, posv):
            base = jnp.minimum(posv, capv - _UB)
            msks = []
            incs = []
            for u in range(_UB):
                c0 = (i * _UB + u) * _L
                x = row_v[pl.ds(c0, _L)]
                msk = x >= t0
                row_v[pl.ds(c0, _L)] = jnp.where(msk, x, ninf)
                msks.append(msk)
                incs.append(msk.astype(jnp.int32))
            run = jnp.zeros((_L,), jnp.int32)
            for u in range(_UB):
                c0 = (i * _UB + u) * _L
                plsc.store_scatter(candi, [base + run], iota + c0,
                                   mask=msks[u])
                run = run + incs[u]
            return base + run

        posv = lax.fori_loop(0, n_chunks // _UB, pass_b, lane_base)
        cnt_lane = jnp.minimum(posv - lane_base, _STRIDE - 1)
        nch_seg = jnp.max(cnt_lane)

        def compact(ch, base):
            iv = plsc.load_gather(candi, [lane_base + ch]) & \
                jnp.int32(_COLS - 1)
            validm = cnt_lane > ch
            sk = _skey(plsc.load_gather(row_v, [iv], mask=validm))
            pc = plsc.cumsum(validm.astype(jnp.int32))
            pos = jnp.minimum(base + pc - 1, _CAP - 1)
            plsc.store_scatter(skbuf, [pos], sk, mask=validm)
            plsc.store_scatter(ivbuf, [pos], iv, mask=validm)
            return base + _splat_last(pc)

        n_splat = lax.fori_loop(0, nch_seg, compact,
                                jnp.zeros((_L,), jnp.int32))
        nch = (jnp.max(n_splat) + _L - 1) // _L

        def valid_at(ch):
            return (iota + ch * _L) < n_splat

        def narrow_chunk(ch, carry):
            m1, m2, m3 = carry
            sk = jnp.where(valid_at(ch), skbuf[pl.ds(ch * _L, _L)],
                           jnp.int32(_MIN32))
            hi1 = jnp.maximum(m1, sk)
            lo1 = jnp.minimum(m1, sk)
            hi2 = jnp.maximum(m2, lo1)
            lo2 = jnp.minimum(m2, lo1)
            return hi1, hi2, jnp.maximum(m3, lo2)

        minv = jnp.full((_L,), _MIN32, jnp.int32)
        _m1, _m2, m3 = lax.fori_loop(0, nch, narrow_chunk,
                                     (minv, minv, minv))
        t1k = _splat_min_i32(m3)

        def recompact(ch, base):
            sk = skbuf[pl.ds(ch * _L, _L)]
            v2 = (sk >= t1k) & valid_at(ch)
            pc = plsc.cumsum(v2.astype(jnp.int32))
            pos = jnp.minimum(base + pc - 1, _CAP2 - 1)
            plsc.store_scatter(skb2, [pos], sk, mask=v2)
            return base + _splat_last(pc)

        n2_splat = lax.fori_loop(0, nch, recompact,
                                 jnp.zeros((_L,), jnp.int32))
        nch2 = (jnp.max(n2_splat) + _L - 1) // _L

        def valid2_at(ch):
            return (iota + ch * _L) < n2_splat

        def search_step(k, prefix_v):
            bit = jnp.int32(1) << (jnp.int32(31) - k)
            cand_v = prefix_v | bit
            cand_s = cand_v ^ jnp.int32(_MIN32)

            def count_chunk(ch, acc):
                sk = skb2[pl.ds(ch * _L, _L)]
                return acc + ((sk >= cand_s) & valid2_at(ch)).astype(
                    jnp.int32)

            acc = lax.fori_loop(0, nch2, count_chunk,
                                jnp.zeros((_L,), jnp.int32))
            tot = _splat_total(acc)
            return jnp.where(tot >= _K, cand_v, prefix_v)

        prefix_v = lax.fori_loop(0, 32, search_step,
                                 jnp.zeros((_L,), jnp.int32))
        t_v = prefix_v ^ jnp.int32(_MIN32)

        def gteq_chunk(ch, carry):
            gt_acc, eq_acc = carry
            sk = skb2[pl.ds(ch * _L, _L)]
            va = valid2_at(ch)
            return (gt_acc + ((sk > t_v) & va).astype(jnp.int32),
                    eq_acc + ((sk == t_v) & va).astype(jnp.int32))

        zero = jnp.zeros((_L,), jnp.int32)
        n_gt, n_eq = lax.fori_loop(0, nch2, gteq_chunk, (zero, zero))
        n_gt = _splat_total(n_gt)
        n_eq = _splat_total(n_eq)
        need = _K - n_gt
        s_rare = jnp.max(n_eq - need)

        @pl.when(s_rare == 0)
        def _():
            def promote_all(ch, _c):
                sk = skbuf[pl.ds(ch * _L, _L)]
                skbuf[pl.ds(ch * _L, _L)] = jnp.where(
                    (sk == t_v) & valid_at(ch), t_v + 1, sk)
                return 0

            lax.fori_loop(0, nch, promote_all, 0)

        @pl.when(s_rare != 0)
        def _():
            def promote_one(_r, _c):
                def min_chunk(ch, macc):
                    sk = skbuf[pl.ds(ch * _L, _L)]
                    ivv = ivbuf[pl.ds(ch * _L, _L)]
                    m = (sk == t_v) & valid_at(ch)
                    return jnp.minimum(
                        macc, jnp.where(m, ivv, jnp.int32(_MAX32)))

                macc = lax.fori_loop(0, nch, min_chunk,
                                     jnp.full((_L,), _MAX32, jnp.int32))
                mins = _splat_min_i32(macc)

                def mark_chunk(ch, _c2):
                    sk = skbuf[pl.ds(ch * _L, _L)]
                    ivv = ivbuf[pl.ds(ch * _L, _L)]
                    match = (ivv == mins) & (sk == t_v) & valid_at(ch)
                    skbuf[pl.ds(ch * _L, _L)] = jnp.where(
                        match, t_v + 1, sk)
                    return 0

                lax.fori_loop(0, nch, mark_chunk, 0)
                return 0

            lax.fori_loop(0, jnp.max(need), promote_one, 0)

        def fixup_chunk(ch, _c):
            sk = skbuf[pl.ds(ch * _L, _L)]
            iv = ivbuf[pl.ds(ch * _L, _L)] & jnp.int32(_COLS - 1)
            rej = valid_at(ch) & (sk <= t_v)
            plsc.store_scatter(row_v, [iv], ninf, mask=rej)
            return 0

        lax.fori_loop(0, nch, fixup_chunk, 0)

    in_copy(0).start()
    for j in range(rows_per):
        if j + 1 < rows_per:
            if j >= 1:
                out_copy(j - 1).wait()
            in_copy(j + 1).start()
        in_copy(j).wait()
        compute(bufs[j % 2])
        out_copy(j).start()
    out_copy(rows_per - 2).wait()
    out_copy(rows_per - 1).wait()


@jax.jit
def kernel(sim):
    mesh = plsc.VectorSubcoreMesh(core_axis_name="c", subcore_axis_name="s")
    f = functools.partial(
        pl.kernel,
        mesh=mesh,
        out_type=jax.ShapeDtypeStruct((_ROWS, _COLS), jnp.float32),
        scratch_types=[
            pltpu.VMEM((_COLS,), jnp.float32),
            pltpu.VMEM((_COLS,), jnp.float32),
            pltpu.VMEM((_CAP,), jnp.int32),
            pltpu.VMEM((_CAP,), jnp.int32),
            pltpu.VMEM((_CAP,), jnp.int32),
            pltpu.VMEM((_CAP2,), jnp.int32),
            pltpu.SemaphoreType.DMA,
            pltpu.SemaphoreType.DMA,
            pltpu.SemaphoreType.DMA,
            pltpu.SemaphoreType.DMA,
        ],
        compiler_params=pltpu.CompilerParams(needs_layout_passes=False),
    )(_sc_body)
    return f(sim)
